# Initial kernel scaffold; baseline (speedup 1.0000x reference)
import jax
import jax.numpy as jnp
from jax.experimental import pallas as pl


def kernel(x_user, x_job, edge_index, rev_edge_index, edge_label_index, W_user, b_user, W_job, b_job, bn_g_user, bn_b_user, bn_g_job, bn_b_job, c1_rates_Wl, c1_rates_bl, c1_rates_Wr, c1_rev_Wl, c1_rev_bl, c1_rev_Wr, c2_rates_Wl, c2_rates_bl, c2_rates_Wr, c2_rev_Wl, c2_rev_bl, c2_rev_Wr):
    raise NotImplementedError("write your pallas kernel here")


# speedup vs baseline: 2.8779x; 2.8779x over previous
import functools

import jax
import jax.numpy as jnp
from jax import lax
from jax.experimental import pallas as pl
from jax.experimental.pallas import tpu as pltpu
from jax.experimental.pallas import tpu_sc as plsc

N = 10000
D = 128
E = 320000
L = 100000
NC, NS, LANES = 2, 16, 16
NW = NC * NS

ROWS_PER_TILE = 632
NPAD = NS * ROWS_PER_TILE
HD = 64
CHUNK = 128
NCHUNK_E = 157
EP = NS * NCHUNK_E * CHUNK
NCHUNK_L = 25
LP = NW * NCHUNK_L * CHUNK


def _make_segsum(with_counts):
    mesh = plsc.VectorSubcoreMesh(core_axis_name="c", subcore_axis_name="s")
    out_type = [jax.ShapeDtypeStruct((NC, NPAD, HD), jnp.float32)]
    scratch = [
        pltpu.VMEM((NCHUNK_E, CHUNK), jnp.int32),
        pltpu.VMEM((NCHUNK_E, CHUNK), jnp.int32),
        pltpu.VMEM((CHUNK, HD), jnp.float32),
        pltpu.VMEM_SHARED((NPAD, HD), jnp.float32),
        pltpu.SemaphoreType.DMA,
    ]
    if with_counts:
        out_type.append(jax.ShapeDtypeStruct((NC, NPAD, LANES), jnp.float32))
        scratch += [
            pltpu.VMEM((CHUNK, LANES), jnp.float32),
            pltpu.VMEM_SHARED((NPAD, LANES), jnp.float32),
        ]

    def body(*args):
        if with_counts:
            (tab, srci, dsti, zf, zc, ones_h, sums, cnt,
             src_v, dst_v, rows_v, acc_sh, sem, ones_v, cnt_sh) = args
        else:
            (tab, srci, dsti, zf, sums,
             src_v, dst_v, rows_v, acc_sh, sem) = args
        cid = lax.axis_index("c")
        sid = lax.axis_index("s")
        base = pl.multiple_of(sid * ROWS_PER_TILE, 8)
        sl = pl.ds(base, ROWS_PER_TILE)
        pltpu.sync_copy(zf.at[sl], acc_sh.at[sl])
        pltpu.sync_copy(srci.at[cid, sid], src_v)
        pltpu.sync_copy(dsti.at[cid, sid], dst_v)
        if with_counts:
            pltpu.sync_copy(zc.at[sl], cnt_sh.at[sl])
            pltpu.sync_copy(ones_h, ones_v)
        plsc.subcore_barrier()

        def step(ci, carry):
            pltpu.async_copy(tab.at[src_v.at[ci]], rows_v, sem).wait()
            pltpu.sync_copy(rows_v, acc_sh.at[dst_v.at[ci]], add=True)
            if with_counts:
                pltpu.sync_copy(ones_v, cnt_sh.at[dst_v.at[ci]], add=True)
            return carry

        lax.fori_loop(0, NCHUNK_E, step, 0)
        plsc.subcore_barrier()
        pltpu.sync_copy(acc_sh.at[sl], sums.at[cid, sl])
        if with_counts:
            pltpu.sync_copy(cnt_sh.at[sl], cnt.at[cid, sl])

    return pl.kernel(body, out_type=tuple(out_type), mesh=mesh,
                     scratch_types=scratch,
                     compiler_params=pltpu.CompilerParams(
                         use_tc_tiling_on_sc=False))


_segsum_wc = _make_segsum(True)
_segsum_nc = _make_segsum(False)


def _make_labels():
    mesh = plsc.VectorSubcoreMesh(core_axis_name="c", subcore_axis_name="s")
    out_type = (jax.ShapeDtypeStruct((LP, D), jnp.float32),
                jax.ShapeDtypeStruct((LP, D), jnp.float32))
    scratch = [
        pltpu.VMEM((NCHUNK_L, CHUNK), jnp.int32),
        pltpu.VMEM((NCHUNK_L, CHUNK), jnp.int32),
        pltpu.VMEM((CHUNK, D), jnp.float32),
        pltpu.VMEM((CHUNK, D), jnp.float32),
        pltpu.SemaphoreType.DMA,
    ]

    def body(tab, l0i, l1i, uf, jf, l0_v, l1_v, rows_u, rows_j, sem):
        cid = lax.axis_index("c")
        sid = lax.axis_index("s")
        w = sid * NC + cid
        pltpu.sync_copy(l0i.at[w], l0_v)
        pltpu.sync_copy(l1i.at[w], l1_v)

        def step(ci, carry):
            rb = pl.multiple_of(w * (NCHUNK_L * CHUNK) + ci * CHUNK, 8)
            pltpu.async_copy(tab.at[l0_v.at[ci]], rows_u, sem).wait()
            pltpu.sync_copy(rows_u, uf.at[pl.ds(rb, CHUNK)])
            pltpu.async_copy(tab.at[l1_v.at[ci]], rows_j, sem).wait()
            pltpu.sync_copy(rows_j, jf.at[pl.ds(rb, CHUNK)])
            return carry

        lax.fori_loop(0, NCHUNK_L, step, 0)

    return pl.kernel(body, out_type=out_type, mesh=mesh,
                     scratch_types=scratch)


_labels = _make_labels()


def _encoder_body(x_ref, w_ref, b_ref, g_ref, bb_ref, o_ref):
    h = jnp.dot(x_ref[...], w_ref[...],
                preferred_element_type=jnp.float32) + b_ref[...]
    hs = h[:N]
    mu = jnp.mean(hs, axis=0, keepdims=True)
    var = jnp.mean((hs - mu) ** 2, axis=0, keepdims=True)
    y = (h - mu) * lax.rsqrt(var + 1e-5) * g_ref[...] + bb_ref[...]
    y = jnp.maximum(y, 0.0)
    rid = lax.broadcasted_iota(jnp.int32, y.shape, 0)
    o_ref[...] = jnp.where(rid < N, y, 0.0)


def _encoder(x, w, b, g, bb):
    return pl.pallas_call(
        _encoder_body,
        out_shape=jax.ShapeDtypeStruct((NPAD, D), jnp.float32),
    )(x, w, b, g, bb)


def _combine_body(relu, sl_ref, sr_ref, c_ref, x_ref, wl_ref, bl_ref,
                  wr_ref, o_ref):
    r = 1.0 / jnp.maximum(c_ref[...][:, 0:1], 1.0)
    agg = jnp.concatenate([sl_ref[...], sr_ref[...]], axis=1) * r
    y = (jnp.dot(agg, wl_ref[...], preferred_element_type=jnp.float32)
         + bl_ref[...]
         + jnp.dot(x_ref[...], wr_ref[...],
                   preferred_element_type=jnp.float32))
    if relu:
        y = jnp.maximum(y, 0.0)
    rid = lax.broadcasted_iota(jnp.int32, y.shape, 0)
    o_ref[...] = jnp.where(rid < N, y, 0.0)


def _combine(relu, s_lo, s_hi, c, x, wl, bl, wr):
    return pl.pallas_call(
        functools.partial(_combine_body, relu),
        out_shape=jax.ShapeDtypeStruct((NPAD, D), jnp.float32),
    )(s_lo, s_hi, c, x, wl, bl, wr)


_DOT_BLK = 2048


def _dot_body(u_ref, j_ref, o_ref):
    o_ref[...] = jnp.sum(u_ref[...] * j_ref[...], axis=1, keepdims=True)


def _dot(uf, jf):
    return pl.pallas_call(
        _dot_body,
        grid=(LP // _DOT_BLK,),
        in_specs=[pl.BlockSpec((_DOT_BLK, D), lambda i: (i, 0)),
                  pl.BlockSpec((_DOT_BLK, D), lambda i: (i, 0))],
        out_specs=pl.BlockSpec((_DOT_BLK, 1), lambda i: (i, 0)),
        out_shape=jax.ShapeDtypeStruct((LP, 1), jnp.float32),
    )(uf, jf)


def kernel(x_user, x_job, edge_index, rev_edge_index, edge_label_index,
           W_user, b_user, W_job, b_job, bn_g_user, bn_b_user, bn_g_job,
           bn_b_job, c1_rates_Wl, c1_rates_bl, c1_rates_Wr, c1_rev_Wl,
           c1_rev_bl, c1_rev_Wr, c2_rates_Wl, c2_rates_bl, c2_rates_Wr,
           c2_rev_Wl, c2_rev_bl, c2_rev_Wr):
    f32 = jnp.float32
    ei = edge_index.astype(jnp.int32)
    rev = rev_edge_index.astype(jnp.int32)
    eli = edge_label_index.astype(jnp.int32)

    xu = jnp.pad(x_user, ((0, NPAD - N), (0, 0)))
    xj = jnp.pad(x_job, ((0, NPAD - N), (0, 0)))
    u = _encoder(xu, W_user, b_user.reshape(1, D), bn_g_user.reshape(1, D),
                 bn_b_user.reshape(1, D))
    j = _encoder(xj, W_job, b_job.reshape(1, D), bn_g_job.reshape(1, D),
                 bn_b_job.reshape(1, D))

    pe = EP - E
    srcA = jnp.concatenate([ei[0], jnp.zeros((pe,), jnp.int32)])
    dstA = jnp.concatenate([ei[1], jnp.full((pe,), N, jnp.int32)])
    srcB = jnp.concatenate([rev[0] + NPAD, jnp.full((pe,), NPAD, jnp.int32)])
    dstB = jnp.concatenate([rev[1], jnp.full((pe,), N, jnp.int32)])
    srci = jnp.stack([srcA, srcB]).reshape(NC, NS, NCHUNK_E, CHUNK)
    dsti = jnp.stack([dstA, dstB]).reshape(NC, NS, NCHUNK_E, CHUNK)

    zf = jnp.zeros((NPAD, HD), f32)
    zc = jnp.zeros((NPAD, LANES), f32)
    ones_h = jnp.ones((CHUNK, LANES), f32)

    tab1 = jnp.concatenate([u, j], axis=0)
    s1lo, cnt = _segsum_wc(tab1[:, :HD], srci, dsti, zf, zc, ones_h)
    (s1hi,) = _segsum_nc(tab1[:, HD:], srci, dsti, zf)
    j1 = _combine(True, s1lo[0], s1hi[0], cnt[0], j, c1_rates_Wl,
                  c1_rates_bl.reshape(1, D), c1_rates_Wr)
    u1 = _combine(True, s1lo[1], s1hi[1], cnt[1], u, c1_rev_Wl,
                  c1_rev_bl.reshape(1, D), c1_rev_Wr)

    tab2 = jnp.concatenate([u1, j1], axis=0)
    (s2lo,) = _segsum_nc(tab2[:, :HD], srci, dsti, zf)
    (s2hi,) = _segsum_nc(tab2[:, HD:], srci, dsti, zf)
    j2 = _combine(False, s2lo[0], s2hi[0], cnt[0], j1, c2_rates_Wl,
                  c2_rates_bl.reshape(1, D), c2_rates_Wr)
    u2 = _combine(False, s2lo[1], s2hi[1], cnt[1], u1, c2_rev_Wl,
                  c2_rev_bl.reshape(1, D), c2_rev_Wr)

    pla = LP - L
    l0 = jnp.concatenate([eli[0], jnp.zeros((pla,), jnp.int32)])
    l1 = jnp.concatenate([eli[1] + NPAD, jnp.full((pla,), NPAD, jnp.int32)])
    tab3 = jnp.concatenate([u2, j2], axis=0)
    uf, jf = _labels(tab3, l0.reshape(NW, NCHUNK_L, CHUNK),
                     l1.reshape(NW, NCHUNK_L, CHUNK))
    dots = _dot(uf, jf)
    return dots[:L, 0]


# speedup vs baseline: 4.2788x; 1.4868x over previous
import functools

import jax
import jax.numpy as jnp
from jax import lax
from jax.experimental import pallas as pl
from jax.experimental.pallas import tpu as pltpu
from jax.experimental.pallas import tpu_sc as plsc

N = 10000
D = 128
E = 320000
L = 100000
NC, NS, LANES = 2, 16, 16
NW = NC * NS

ROWS_PER_TILE = 632
NPAD = NS * ROWS_PER_TILE
HD = 64
CHUNK = 128
NCHUNK_E = 157
EP = NS * NCHUNK_E * CHUNK
NCHUNK_L = 25
LP = NW * NCHUNK_L * CHUNK


def _make_segsum(with_counts):
    mesh = plsc.VectorSubcoreMesh(core_axis_name="c", subcore_axis_name="s")
    out_type = [jax.ShapeDtypeStruct((NC, NPAD, HD), jnp.float32)]
    scratch = [
        pltpu.VMEM((NCHUNK_E, CHUNK), jnp.int32),
        pltpu.VMEM((NCHUNK_E, CHUNK), jnp.int32),
    ]
    scratch += [pltpu.VMEM((CHUNK, HD), jnp.float32) for _ in range(4)]
    scratch += [
        pltpu.VMEM_SHARED((NPAD, HD), jnp.float32),
        pltpu.SemaphoreType.DMA((4,)),
        pltpu.SemaphoreType.DMA((4,)),
    ]
    if with_counts:
        out_type.append(jax.ShapeDtypeStruct((NC, NPAD, LANES), jnp.float32))
        scratch += [
            pltpu.VMEM((CHUNK, LANES), jnp.float32),
            pltpu.VMEM_SHARED((NPAD, LANES), jnp.float32),
            pltpu.SemaphoreType.DMA((4,)),
        ]
    LAST = NCHUNK_E - 1

    def body(*args):
        if with_counts:
            (tab, srci, dsti, zf, zc, ones_h, sums, cnt, src_v, dst_v,
             b0, b1, b2, b3, acc_sh, gsem, ssem,
             ones_v, cnt_sh, csem) = args
        else:
            (tab, srci, dsti, zf, sums, src_v, dst_v,
             b0, b1, b2, b3, acc_sh, gsem, ssem) = args
        bufs = (b0, b1, b2, b3)
        cid = lax.axis_index("c")
        sid = lax.axis_index("s")
        base = pl.multiple_of(sid * ROWS_PER_TILE, 8)
        sl = pl.ds(base, ROWS_PER_TILE)
        pltpu.sync_copy(zf.at[sl], acc_sh.at[sl])
        pltpu.sync_copy(srci.at[cid, sid], src_v)
        pltpu.sync_copy(dsti.at[cid, sid], dst_v)
        if with_counts:
            pltpu.sync_copy(zc.at[sl], cnt_sh.at[sl])
            pltpu.sync_copy(ones_h, ones_v)
        plsc.subcore_barrier()

        def g_wait(b):
            pltpu.make_async_copy(tab.at[pl.ds(0, CHUNK)], bufs[b],
                                  gsem.at[b]).wait()

        def s_wait(b):
            pltpu.make_async_copy(tab.at[pl.ds(0, CHUNK)], bufs[b],
                                  ssem.at[b]).wait()

        def c_wait(b):
            pltpu.make_async_copy(zc.at[pl.ds(0, CHUNK)], ones_v,
                                  csem.at[b]).wait()

        def g_fire(ci, b):
            pltpu.async_copy(tab.at[src_v.at[ci]], bufs[b], gsem.at[b])

        def s_fire(ci, b):
            pltpu.async_copy(bufs[b], acc_sh.at[dst_v.at[ci]], ssem.at[b],
                             add=True)
            if with_counts:
                pltpu.async_copy(ones_v, cnt_sh.at[dst_v.at[ci]],
                                 csem.at[b], add=True)

        g_fire(0, 0)
        g_fire(1, 1)

        def outer(io, carry):
            i = io * 4
            for b in range(4):
                ci = i + b
                nb = (b + 2) % 4
                g_wait(b)
                s_fire(ci, b)

                @pl.when(ci >= 2)
                def _():
                    s_wait(nb)
                    if with_counts:
                        c_wait(nb)

                @pl.when(ci <= LAST - 2)
                def _():
                    g_fire(ci + 2, nb)
            return carry

        lax.fori_loop(0, LAST // 4, outer, 0)
        g_wait(0)
        s_fire(LAST, 0)
        for b in (2, 3, 0):
            s_wait(b)
            if with_counts:
                c_wait(b)
        plsc.subcore_barrier()
        pltpu.sync_copy(acc_sh.at[sl], sums.at[cid, sl])
        if with_counts:
            pltpu.sync_copy(cnt_sh.at[sl], cnt.at[cid, sl])

    return pl.kernel(body, out_type=tuple(out_type), mesh=mesh,
                     scratch_types=scratch,
                     compiler_params=pltpu.CompilerParams(
                         use_tc_tiling_on_sc=False))


_segsum_wc = _make_segsum(True)
_segsum_nc = _make_segsum(False)


def _make_labels():
    mesh = plsc.VectorSubcoreMesh(core_axis_name="c", subcore_axis_name="s")
    out_type = (jax.ShapeDtypeStruct((LP, D), jnp.float32),
                jax.ShapeDtypeStruct((LP, D), jnp.float32))
    scratch = [
        pltpu.VMEM((NCHUNK_L, CHUNK), jnp.int32),
        pltpu.VMEM((NCHUNK_L, CHUNK), jnp.int32),
        pltpu.VMEM((CHUNK, D), jnp.float32),
        pltpu.VMEM((CHUNK, D), jnp.float32),
        pltpu.VMEM((CHUNK, D), jnp.float32),
        pltpu.VMEM((CHUNK, D), jnp.float32),
        pltpu.SemaphoreType.DMA((2,)),
        pltpu.SemaphoreType.DMA((2,)),
        pltpu.SemaphoreType.DMA((2,)),
        pltpu.SemaphoreType.DMA((2,)),
    ]
    LASTL = NCHUNK_L - 1

    def body(tab, l0i, l1i, uf, jf, l0_v, l1_v, ru0, ru1, rj0, rj1,
             gusem, gjsem, wusem, wjsem):
        cid = lax.axis_index("c")
        sid = lax.axis_index("s")
        w = sid * NC + cid
        rus = (ru0, ru1)
        rjs = (rj0, rj1)
        pltpu.sync_copy(l0i.at[w], l0_v)
        pltpu.sync_copy(l1i.at[w], l1_v)

        def g_fire(ci, b):
            pltpu.async_copy(tab.at[l0_v.at[ci]], rus[b], gusem.at[b])
            pltpu.async_copy(tab.at[l1_v.at[ci]], rjs[b], gjsem.at[b])

        def g_wait(b):
            pltpu.make_async_copy(tab.at[pl.ds(0, CHUNK)], rus[b],
                                  gusem.at[b]).wait()
            pltpu.make_async_copy(tab.at[pl.ds(0, CHUNK)], rjs[b],
                                  gjsem.at[b]).wait()

        def w_fire(ci, b):
            rb = pl.multiple_of(w * (NCHUNK_L * CHUNK) + ci * CHUNK, 8)
            pltpu.async_copy(rus[b], uf.at[pl.ds(rb, CHUNK)], wusem.at[b])
            pltpu.async_copy(rjs[b], jf.at[pl.ds(rb, CHUNK)], wjsem.at[b])

        def w_wait(b):
            pltpu.make_async_copy(tab.at[pl.ds(0, CHUNK)], rus[b],
                                  wusem.at[b]).wait()
            pltpu.make_async_copy(tab.at[pl.ds(0, CHUNK)], rjs[b],
                                  wjsem.at[b]).wait()

        g_fire(0, 0)

        def outer(io, carry):
            i = io * 2
            for b in range(2):
                ci = i + b
                nb = 1 - b
                g_wait(b)
                w_fire(ci, b)

                @pl.when(ci >= 1)
                def _():
                    w_wait(nb)

                g_fire(ci + 1, nb)
            return carry

        lax.fori_loop(0, LASTL // 2, outer, 0)
        g_wait(0)
        w_fire(LASTL, 0)
        w_wait(1)
        w_wait(0)

    return pl.kernel(body, out_type=out_type, mesh=mesh,
                     scratch_types=scratch)


_labels = _make_labels()


def _encoder_body(x_ref, w_ref, b_ref, g_ref, bb_ref, o_ref):
    h = jnp.dot(x_ref[...], w_ref[...],
                preferred_element_type=jnp.float32) + b_ref[...]
    hs = h[:N]
    mu = jnp.mean(hs, axis=0, keepdims=True)
    var = jnp.mean((hs - mu) ** 2, axis=0, keepdims=True)
    y = (h - mu) * lax.rsqrt(var + 1e-5) * g_ref[...] + bb_ref[...]
    y = jnp.maximum(y, 0.0)
    rid = lax.broadcasted_iota(jnp.int32, y.shape, 0)
    o_ref[...] = jnp.where(rid < N, y, 0.0)


def _encoder(x, w, b, g, bb):
    return pl.pallas_call(
        _encoder_body,
        out_shape=jax.ShapeDtypeStruct((NPAD, D), jnp.float32),
    )(x, w, b, g, bb)


def _combine_body(relu, sl_ref, sr_ref, c_ref, x_ref, wl_ref, bl_ref,
                  wr_ref, o_ref):
    r = 1.0 / jnp.maximum(c_ref[...][:, 0:1], 1.0)
    agg = jnp.concatenate([sl_ref[...], sr_ref[...]], axis=1) * r
    y = (jnp.dot(agg, wl_ref[...], preferred_element_type=jnp.float32)
         + bl_ref[...]
         + jnp.dot(x_ref[...], wr_ref[...],
                   preferred_element_type=jnp.float32))
    if relu:
        y = jnp.maximum(y, 0.0)
    rid = lax.broadcasted_iota(jnp.int32, y.shape, 0)
    o_ref[...] = jnp.where(rid < N, y, 0.0)


def _combine(relu, s_lo, s_hi, c, x, wl, bl, wr):
    return pl.pallas_call(
        functools.partial(_combine_body, relu),
        out_shape=jax.ShapeDtypeStruct((NPAD, D), jnp.float32),
    )(s_lo, s_hi, c, x, wl, bl, wr)


_DOT_BLK = 2048


def _dot_body(u_ref, j_ref, o_ref):
    o_ref[...] = jnp.sum(u_ref[...] * j_ref[...], axis=1, keepdims=True)


def _dot(uf, jf):
    return pl.pallas_call(
        _dot_body,
        grid=(LP // _DOT_BLK,),
        in_specs=[pl.BlockSpec((_DOT_BLK, D), lambda i: (i, 0)),
                  pl.BlockSpec((_DOT_BLK, D), lambda i: (i, 0))],
        out_specs=pl.BlockSpec((_DOT_BLK, 1), lambda i: (i, 0)),
        out_shape=jax.ShapeDtypeStruct((LP, 1), jnp.float32),
    )(uf, jf)


def kernel(x_user, x_job, edge_index, rev_edge_index, edge_label_index,
           W_user, b_user, W_job, b_job, bn_g_user, bn_b_user, bn_g_job,
           bn_b_job, c1_rates_Wl, c1_rates_bl, c1_rates_Wr, c1_rev_Wl,
           c1_rev_bl, c1_rev_Wr, c2_rates_Wl, c2_rates_bl, c2_rates_Wr,
           c2_rev_Wl, c2_rev_bl, c2_rev_Wr):
    f32 = jnp.float32
    ei = edge_index.astype(jnp.int32)
    rev = rev_edge_index.astype(jnp.int32)
    eli = edge_label_index.astype(jnp.int32)

    xu = jnp.pad(x_user, ((0, NPAD - N), (0, 0)))
    xj = jnp.pad(x_job, ((0, NPAD - N), (0, 0)))
    u = _encoder(xu, W_user, b_user.reshape(1, D), bn_g_user.reshape(1, D),
                 bn_b_user.reshape(1, D))
    j = _encoder(xj, W_job, b_job.reshape(1, D), bn_g_job.reshape(1, D),
                 bn_b_job.reshape(1, D))

    pe = EP - E
    srcA = jnp.concatenate([ei[0], jnp.zeros((pe,), jnp.int32)])
    dstA = jnp.concatenate([ei[1], jnp.full((pe,), N, jnp.int32)])
    srcB = jnp.concatenate([rev[0] + NPAD, jnp.full((pe,), NPAD, jnp.int32)])
    dstB = jnp.concatenate([rev[1], jnp.full((pe,), N, jnp.int32)])
    srci = jnp.stack([srcA, srcB]).reshape(NC, NS, NCHUNK_E, CHUNK)
    dsti = jnp.stack([dstA, dstB]).reshape(NC, NS, NCHUNK_E, CHUNK)

    zf = jnp.zeros((NPAD, HD), f32)
    zc = jnp.zeros((NPAD, LANES), f32)
    ones_h = jnp.ones((CHUNK, LANES), f32)

    tab1 = jnp.concatenate([u, j], axis=0)
    s1lo, cnt = _segsum_wc(tab1[:, :HD], srci, dsti, zf, zc, ones_h)
    (s1hi,) = _segsum_nc(tab1[:, HD:], srci, dsti, zf)
    j1 = _combine(True, s1lo[0], s1hi[0], cnt[0], j, c1_rates_Wl,
                  c1_rates_bl.reshape(1, D), c1_rates_Wr)
    u1 = _combine(True, s1lo[1], s1hi[1], cnt[1], u, c1_rev_Wl,
                  c1_rev_bl.reshape(1, D), c1_rev_Wr)

    tab2 = jnp.concatenate([u1, j1], axis=0)
    (s2lo,) = _segsum_nc(tab2[:, :HD], srci, dsti, zf)
    (s2hi,) = _segsum_nc(tab2[:, HD:], srci, dsti, zf)
    j2 = _combine(False, s2lo[0], s2hi[0], cnt[0], j1, c2_rates_Wl,
                  c2_rates_bl.reshape(1, D), c2_rates_Wr)
    u2 = _combine(False, s2lo[1], s2hi[1], cnt[1], u1, c2_rev_Wl,
                  c2_rev_bl.reshape(1, D), c2_rev_Wr)

    pla = LP - L
    l0 = jnp.concatenate([eli[0], jnp.zeros((pla,), jnp.int32)])
    l1 = jnp.concatenate([eli[1] + NPAD, jnp.full((pla,), NPAD, jnp.int32)])
    tab3 = jnp.concatenate([u2, j2], axis=0)
    uf, jf = _labels(tab3, l0.reshape(NW, NCHUNK_L, CHUNK),
                     l1.reshape(NW, NCHUNK_L, CHUNK))
    dots = _dot(uf, jf)
    return dots[:L, 0]
